# baseline (speedup 1.0000x reference)
import functools

import jax
import jax.numpy as jnp
from jax import lax
from jax.experimental import pallas as pl
from jax.experimental.pallas import tpu as pltpu, tpu_sc as plsc

N = 10000
NPAD = 10240
E = 320000
D = 128
DH = 64
DE = 16

NC = 2
NS = 16
NW = NC * NS
EDGES_PER_TILE = E // NS
SUB = 80
NSUB = EDGES_PER_TILE // SUB
U = 10
NBODY = NSUB // U
ROWS_PER_SUB = NPAD // NS
SUB_A = 40
U_A = 5
NBODY_A = E // (NW * U_A * SUB_A)
NPAD_A = 10112
ROWS_A = NPAD_A // NS

_MESH = plsc.VectorSubcoreMesh(core_axis_name="c", subcore_axis_name="s",
                               num_cores=NC, num_subcores=NS)
_SC_PARAMS = pltpu.CompilerParams(use_tc_tiling_on_sc=False)


UH = U // 2


def _sc_node_path(src4d, dst4d, nf, z128):
  @functools.partial(
      pl.kernel,
      out_type=jax.ShapeDtypeStruct((NC, NPAD_A, D), jnp.float32),
      mesh=_MESH,
      compiler_params=_SC_PARAMS,
      scratch_types=[
          pltpu.VMEM((U_A, SUB_A), jnp.int32),
          pltpu.VMEM((U_A, SUB_A), jnp.int32),
          pltpu.VMEM((U_A, SUB_A, D), jnp.float32),
          pltpu.VMEM_SHARED((NPAD_A, D), jnp.float32),
          pltpu.SemaphoreType.DMA,
          pltpu.SemaphoreType.DMA,
          pltpu.SemaphoreType.DMA,
          pltpu.SemaphoreType.DMA,
          pltpu.SemaphoreType.DMA,
      ],
  )
  def k(src_hbm, dst_hbm, nf_hbm, z128_hbm, xs_out,
        src_v, dst_v, rows_v, xs_sh, isem, gsemA, gsemB, ssemA, ssemB):
    c = lax.axis_index("c")
    s = lax.axis_index("s")
    w = c * NS + s

    r0 = s * ROWS_A
    pltpu.sync_copy(z128_hbm, xs_sh.at[pl.ds(r0, ROWS_A)])
    plsc.subcore_barrier()

    def body(i, carry):
      lds = [pltpu.async_copy(src_hbm.at[w, i], src_v, isem),
             pltpu.async_copy(dst_hbm.at[w, i], dst_v, isem)]
      for d in lds:
        d.wait()
      gA = [pltpu.async_copy(nf_hbm.at[src_v.at[u]], rows_v.at[u], gsemA)
            for u in range(2)]
      gB = [pltpu.async_copy(nf_hbm.at[src_v.at[u]], rows_v.at[u], gsemB)
            for u in range(2, U_A)]
      for d in gA:
        d.wait()
      sA = [pltpu.async_copy(rows_v.at[u], xs_sh.at[dst_v.at[u]],
                             ssemA, add=True)
            for u in range(2)]
      for d in gB:
        d.wait()
      sB = [pltpu.async_copy(rows_v.at[u], xs_sh.at[dst_v.at[u]],
                             ssemB, add=True)
            for u in range(2, U_A)]
      for d in sA:
        d.wait()
      for d in sB:
        d.wait()
      return carry

    lax.fori_loop(0, NBODY_A, body, 0)
    plsc.subcore_barrier()
    pltpu.sync_copy(xs_sh.at[pl.ds(r0, ROWS_A)],
                    xs_out.at[c, pl.ds(r0, ROWS_A)])

  return k(src4d, dst4d, nf, z128)


def _sc_edge_path(dst4d, ef, z16, ones16):
  @functools.partial(
      pl.kernel,
      out_type=[
          jax.ShapeDtypeStruct((NC, NPAD, DE), jnp.float32),
          jax.ShapeDtypeStruct((NC, NPAD, DE), jnp.float32),
      ],
      mesh=_MESH,
      compiler_params=_SC_PARAMS,
      scratch_types=[
          pltpu.VMEM((U, SUB), jnp.int32),
          pltpu.VMEM((U, SUB, DE), jnp.float32),
          pltpu.VMEM((SUB, DE), jnp.float32),
          pltpu.VMEM_SHARED((NPAD, DE), jnp.float32),
          pltpu.VMEM_SHARED((NPAD, DE), jnp.float32),
          pltpu.SemaphoreType.DMA,
          pltpu.SemaphoreType.DMA,
          pltpu.SemaphoreType.DMA,
          pltpu.SemaphoreType.DMA,
      ],
  )
  def k(dst_hbm, ef_hbm, z16_hbm, ones_hbm, es_out, deg_out,
        dst_v, ef_v, ones_v, es_sh, deg_sh, isem, esem, asem, dsem):
    c = lax.axis_index("c")
    s = lax.axis_index("s")

    r0 = s * ROWS_PER_SUB
    pltpu.sync_copy(z16_hbm, es_sh.at[pl.ds(r0, ROWS_PER_SUB)])
    pltpu.sync_copy(z16_hbm, deg_sh.at[pl.ds(r0, ROWS_PER_SUB)])
    pltpu.sync_copy(ones_hbm, ones_v)
    plsc.subcore_barrier()

    def body(i, carry):
      ib = 2 * i + c

      @pl.when(ib < NBODY)
      def _():
        ld = pltpu.async_copy(dst_hbm.at[s, ib], dst_v, isem)
        eds = [pltpu.async_copy(
                   ef_hbm.at[pl.ds((s * NSUB + ib * U + u) * SUB, SUB)],
                   ef_v.at[u], esem)
               for u in range(U)]
        ld.wait()
        dds = [pltpu.async_copy(ones_v, deg_sh.at[dst_v.at[u]],
                                dsem, add=True)
               for u in range(U)]
        for d in eds:
          d.wait()
        ads = [pltpu.async_copy(ef_v.at[u], es_sh.at[dst_v.at[u]],
                                asem, add=True)
               for u in range(U)]
        for d in dds:
          d.wait()
        for d in ads:
          d.wait()

      return carry

    lax.fori_loop(0, (NBODY + 1) // 2, body, 0)
    plsc.subcore_barrier()
    pltpu.sync_copy(es_sh.at[pl.ds(r0, ROWS_PER_SUB)],
                    es_out.at[c, pl.ds(r0, ROWS_PER_SUB)])
    pltpu.sync_copy(deg_sh.at[pl.ds(r0, ROWS_PER_SUB)],
                    deg_out.at[c, pl.ds(r0, ROWS_PER_SUB)])

  return k(dst4d, ef, z16, ones16)


_R = 2000


def _tc_body(xs_ref, es_ref, deg_ref, Wn_ref, We_ref, Wm_ref,
             bn_ref, be_ref, bm_ref, out_ref):
  f32 = jnp.float32
  hi = lax.Precision.HIGHEST
  Wm = Wm_ref[...]
  Wq = lax.dot_general(Wm, Wn_ref[...], (((1,), (0,)), ((), ())),
                       precision=hi, preferred_element_type=f32)
  Wr = lax.dot_general(Wm, We_ref[...], (((1,), (0,)), ((), ())),
                       precision=hi, preferred_element_type=f32)
  bsum = (bn_ref[...] + be_ref[...])[None, :]
  cvec = lax.dot_general(bsum, Wm, (((1,), (1,)), ((), ())),
                         precision=hi, preferred_element_type=f32)
  cvec = cvec + bm_ref[...][None, :]

  es = es_ref[0] + es_ref[1]
  deg = jnp.max(deg_ref[0] + deg_ref[1], axis=1, keepdims=True)

  xs = xs_ref[0] + xs_ref[1]
  num = lax.dot_general(xs, Wq, (((1,), (1,)), ((), ())),
                        precision=hi, preferred_element_type=f32)
  num += lax.dot_general(es, Wr, (((1,), (1,)), ((), ())),
                         precision=hi, preferred_element_type=f32)
  num += deg * cvec
  out_ref[...] = num / jnp.maximum(deg, 1.0)


def kernel(node_features, edge_features, edge_index, Wn, bn, We, be, Wm, bm):
  src4a = edge_index[0].reshape(NW, NBODY_A, U_A, SUB_A)
  dst4a = edge_index[1].reshape(NW, NBODY_A, U_A, SUB_A)
  dst4b = edge_index[1].reshape(NS, NBODY, U, SUB)
  z128 = jnp.zeros((ROWS_A, D), jnp.float32)
  z16 = jnp.zeros((ROWS_PER_SUB, DE), jnp.float32)
  ones16 = jnp.ones((SUB, DE), jnp.float32)

  xs_p = _sc_node_path(src4a, dst4a, node_features, z128)
  es_p, deg_p = _sc_edge_path(dst4b, edge_features, z16, ones16)

  out = pl.pallas_call(
      _tc_body,
      grid=(N // _R,),
      in_specs=[
          pl.BlockSpec((NC, _R, D), lambda i: (0, i, 0)),
          pl.BlockSpec((NC, _R, DE), lambda i: (0, i, 0)),
          pl.BlockSpec((NC, _R, DE), lambda i: (0, i, 0)),
          pl.BlockSpec((D, D), lambda i: (0, 0)),
          pl.BlockSpec((D, DE), lambda i: (0, 0)),
          pl.BlockSpec((D, D), lambda i: (0, 0)),
          pl.BlockSpec((D,), lambda i: (0,)),
          pl.BlockSpec((D,), lambda i: (0,)),
          pl.BlockSpec((D,), lambda i: (0,)),
      ],
      out_specs=pl.BlockSpec((_R, D), lambda i: (i, 0)),
      out_shape=jax.ShapeDtypeStruct((N, D), jnp.float32),
  )(xs_p, es_p, deg_p, Wn, We, Wm, bn, be, bm)
  return out


# speedup vs baseline: 1.4121x; 1.4121x over previous
import functools

import jax
import jax.numpy as jnp
from jax import lax
from jax.experimental import pallas as pl
from jax.experimental.pallas import tpu as pltpu, tpu_sc as plsc

N = 10000
NPAD = 10240
E = 320000
D = 128
DH = 64
DE = 16

NC = 2
NS = 16
EDGES_PER_TILE = E // NS
SUB = 80
NSUB = EDGES_PER_TILE // SUB
U = 10
NBODY = NSUB // U
ROWS_PER_SUB = NPAD // NS

_MESH = plsc.VectorSubcoreMesh(core_axis_name="c", subcore_axis_name="s",
                               num_cores=NC, num_subcores=NS)
_SC_PARAMS = pltpu.CompilerParams(use_tc_tiling_on_sc=False)


UH = U // 2


def _sc_node_path(src4d, dst4d, nf2, z64, z16, ones16):
  @functools.partial(
      pl.kernel,
      out_type=[
          jax.ShapeDtypeStruct((NC, NPAD, DH), jnp.float32),
          jax.ShapeDtypeStruct((NC, NPAD, DE), jnp.float32),
      ],
      mesh=_MESH,
      compiler_params=_SC_PARAMS,
      scratch_types=[
          pltpu.VMEM((NSUB, SUB), jnp.int32),
          pltpu.VMEM((U, SUB), jnp.int32),
          pltpu.VMEM((U, SUB, DH), jnp.float32),
          pltpu.VMEM((SUB, DE), jnp.float32),
          pltpu.VMEM_SHARED((NPAD, DH), jnp.float32),
          pltpu.VMEM_SHARED((NPAD, DE), jnp.float32),
          pltpu.SemaphoreType.DMA,
          pltpu.SemaphoreType.DMA,
          pltpu.SemaphoreType.DMA,
          pltpu.SemaphoreType.DMA,
          pltpu.SemaphoreType.DMA,
          pltpu.SemaphoreType.DMA,
      ],
  )
  def k(src_hbm, dst_hbm, nf2_hbm, z64_hbm, z16_hbm, ones_hbm,
        xs_out, deg_out,
        src_v, dst_v, rows_v, ones_v, xs_sh, deg_sh,
        isem, gsemA, gsemB, ssemA, ssemB, asem):
    c = lax.axis_index("c")
    s = lax.axis_index("s")

    r0 = s * ROWS_PER_SUB
    pltpu.sync_copy(z64_hbm, xs_sh.at[pl.ds(r0, ROWS_PER_SUB)])
    pltpu.sync_copy(z16_hbm, deg_sh.at[pl.ds(r0, ROWS_PER_SUB)])
    pltpu.sync_copy(ones_hbm, ones_v)
    pltpu.sync_copy(src_hbm.at[s], src_v)

    def mk_idx(j, carry):
      for kk in range(SUB // 16):
        sl = pl.ds(kk * 16, 16)
        src_v[j, sl] = src_v[j, sl] * 2 + c
      return carry

    lax.fori_loop(0, NSUB, mk_idx, 0)
    plsc.subcore_barrier()

    def body(i, carry):
      dld = pltpu.async_copy(dst_hbm.at[s, i], dst_v, isem)
      gA = [pltpu.async_copy(nf2_hbm.at[src_v.at[i * U + u]],
                             rows_v.at[u], gsemA)
            for u in range(UH)]
      gB = [pltpu.async_copy(nf2_hbm.at[src_v.at[i * U + u]],
                             rows_v.at[u], gsemB)
            for u in range(UH, U)]
      dld.wait()
      for d in gA:
        d.wait()

      sA = [pltpu.async_copy(rows_v.at[u], xs_sh.at[dst_v.at[u]],
                             ssemA, add=True)
            for u in range(UH)]

      @pl.when(c == 0)
      def _():
        dds = [pltpu.async_copy(ones_v, deg_sh.at[dst_v.at[u]],
                                asem, add=True)
               for u in range(UH)]
        for d in dds:
          d.wait()

      @pl.when(c == 1)
      def _():
        dds = [pltpu.async_copy(ones_v, deg_sh.at[dst_v.at[u]],
                                asem, add=True)
               for u in range(UH, U)]
        for d in dds:
          d.wait()

      for d in gB:
        d.wait()
      sB = [pltpu.async_copy(rows_v.at[u], xs_sh.at[dst_v.at[u]],
                             ssemB, add=True)
            for u in range(UH, U)]
      for d in sA:
        d.wait()
      for d in sB:
        d.wait()
      return carry

    lax.fori_loop(0, NBODY, body, 0)
    plsc.subcore_barrier()
    pltpu.sync_copy(xs_sh.at[pl.ds(r0, ROWS_PER_SUB)],
                    xs_out.at[c, pl.ds(r0, ROWS_PER_SUB)])
    pltpu.sync_copy(deg_sh.at[pl.ds(r0, ROWS_PER_SUB)],
                    deg_out.at[c, pl.ds(r0, ROWS_PER_SUB)])

  return k(src4d, dst4d, nf2, z64, z16, ones16)


def _sc_edge_path(dst4d, ef, z16):
  @functools.partial(
      pl.kernel,
      out_type=jax.ShapeDtypeStruct((NC, NPAD, DE), jnp.float32),
      mesh=_MESH,
      compiler_params=_SC_PARAMS,
      scratch_types=[
          pltpu.VMEM((U, SUB), jnp.int32),
          pltpu.VMEM((U, SUB, DE), jnp.float32),
          pltpu.VMEM_SHARED((NPAD, DE), jnp.float32),
          pltpu.SemaphoreType.DMA,
          pltpu.SemaphoreType.DMA,
          pltpu.SemaphoreType.DMA,
      ],
  )
  def k(dst_hbm, ef_hbm, z16_hbm, es_out,
        dst_v, ef_v, es_sh, isem, esem, asem):
    c = lax.axis_index("c")
    s = lax.axis_index("s")

    r0 = s * ROWS_PER_SUB
    pltpu.sync_copy(z16_hbm, es_sh.at[pl.ds(r0, ROWS_PER_SUB)])
    plsc.subcore_barrier()

    def body(i, carry):
      ib = 2 * i + c

      @pl.when(ib < NBODY)
      def _():
        ld = pltpu.async_copy(dst_hbm.at[s, ib], dst_v, isem)
        eds = [pltpu.async_copy(
                   ef_hbm.at[pl.ds((s * NSUB + ib * U + u) * SUB, SUB)],
                   ef_v.at[u], esem)
               for u in range(U)]
        ld.wait()
        for d in eds:
          d.wait()
        ads = [pltpu.async_copy(ef_v.at[u], es_sh.at[dst_v.at[u]],
                                asem, add=True)
               for u in range(U)]
        for d in ads:
          d.wait()

      return carry

    lax.fori_loop(0, (NBODY + 1) // 2, body, 0)
    plsc.subcore_barrier()
    pltpu.sync_copy(es_sh.at[pl.ds(r0, ROWS_PER_SUB)],
                    es_out.at[c, pl.ds(r0, ROWS_PER_SUB)])

  return k(dst4d, ef, z16)


_R = 2000


def _tc_body(xs_ref, es_ref, deg_ref, Wn_ref, We_ref, Wm_ref,
             bn_ref, be_ref, bm_ref, out_ref):
  f32 = jnp.float32
  hi = lax.Precision.HIGHEST
  Wm = Wm_ref[...]
  Wq = lax.dot_general(Wm, Wn_ref[...], (((1,), (0,)), ((), ())),
                       precision=hi, preferred_element_type=f32)
  Wr = lax.dot_general(Wm, We_ref[...], (((1,), (0,)), ((), ())),
                       precision=hi, preferred_element_type=f32)
  bsum = (bn_ref[...] + be_ref[...])[None, :]
  cvec = lax.dot_general(bsum, Wm, (((1,), (1,)), ((), ())),
                         precision=hi, preferred_element_type=f32)
  cvec = cvec + bm_ref[...][None, :]

  es = es_ref[0] + es_ref[1]
  deg = jnp.max(deg_ref[0] + deg_ref[1], axis=1, keepdims=True)

  num = lax.dot_general(xs_ref[0], Wq[:, :DH], (((1,), (1,)), ((), ())),
                        precision=hi, preferred_element_type=f32)
  num += lax.dot_general(xs_ref[1], Wq[:, DH:], (((1,), (1,)), ((), ())),
                         precision=hi, preferred_element_type=f32)
  num += lax.dot_general(es, Wr, (((1,), (1,)), ((), ())),
                         precision=hi, preferred_element_type=f32)
  num += deg * cvec
  out_ref[...] = num / jnp.maximum(deg, 1.0)


def kernel(node_features, edge_features, edge_index, Wn, bn, We, be, Wm, bm):
  src3d = edge_index[0].reshape(NS, NSUB, SUB)
  dst4d = edge_index[1].reshape(NS, NBODY, U, SUB)
  nf2 = node_features.reshape(2 * N, DH)
  z64 = jnp.zeros((ROWS_PER_SUB, DH), jnp.float32)
  z16 = jnp.zeros((ROWS_PER_SUB, DE), jnp.float32)
  ones16 = jnp.ones((SUB, DE), jnp.float32)

  xs_p, deg_p = _sc_node_path(src3d, dst4d, nf2, z64, z16, ones16)
  es_p = _sc_edge_path(dst4d, edge_features, z16)

  out = pl.pallas_call(
      _tc_body,
      grid=(N // _R,),
      in_specs=[
          pl.BlockSpec((NC, _R, DH), lambda i: (0, i, 0)),
          pl.BlockSpec((NC, _R, DE), lambda i: (0, i, 0)),
          pl.BlockSpec((NC, _R, DE), lambda i: (0, i, 0)),
          pl.BlockSpec((D, D), lambda i: (0, 0)),
          pl.BlockSpec((D, DE), lambda i: (0, 0)),
          pl.BlockSpec((D, D), lambda i: (0, 0)),
          pl.BlockSpec((D,), lambda i: (0,)),
          pl.BlockSpec((D,), lambda i: (0,)),
          pl.BlockSpec((D,), lambda i: (0,)),
      ],
      out_specs=pl.BlockSpec((_R, D), lambda i: (i, 0)),
      out_shape=jax.ShapeDtypeStruct((N, D), jnp.float32),
  )(xs_p, es_p, deg_p, Wn, We, Wm, bn, be, bm)
  return out


# speedup vs baseline: 1.4420x; 1.0212x over previous
import functools

import jax
import jax.numpy as jnp
from jax import lax
from jax.experimental import pallas as pl
from jax.experimental.pallas import tpu as pltpu, tpu_sc as plsc

N = 10000
NPAD = 10240
E = 320000
D = 128
DH = 64
DE = 16

NC = 2
NS = 16
EDGES_PER_TILE = E // NS
SUB = 80
NSUB = EDGES_PER_TILE // SUB
U = 10
NBODY = NSUB // U
ROWS_PER_SUB = NPAD // NS

_MESH = plsc.VectorSubcoreMesh(core_axis_name="c", subcore_axis_name="s",
                               num_cores=NC, num_subcores=NS)
_SC_PARAMS = pltpu.CompilerParams(use_tc_tiling_on_sc=False)


UH = U // 2


def _sc_node_path(src4d, dst4d, nf2, z64, z16, ones16):
  @functools.partial(
      pl.kernel,
      out_type=[
          jax.ShapeDtypeStruct((NC, NPAD, DH), jnp.float32),
          jax.ShapeDtypeStruct((NC, NPAD, DE), jnp.float32),
      ],
      mesh=_MESH,
      compiler_params=_SC_PARAMS,
      scratch_types=[
          pltpu.VMEM((NSUB, SUB), jnp.int32),
          pltpu.VMEM((U, SUB), jnp.int32),
          pltpu.VMEM((U, SUB, DH), jnp.float32),
          pltpu.VMEM((SUB, DE), jnp.float32),
          pltpu.VMEM_SHARED((NPAD, DH), jnp.float32),
          pltpu.VMEM_SHARED((NPAD, DE), jnp.float32),
          pltpu.SemaphoreType.DMA,
          pltpu.SemaphoreType.DMA,
          pltpu.SemaphoreType.DMA,
          pltpu.SemaphoreType.DMA,
          pltpu.SemaphoreType.DMA,
          pltpu.SemaphoreType.DMA,
      ],
  )
  def k(src_hbm, dst_hbm, nf2_hbm, z64_hbm, z16_hbm, ones_hbm,
        xs_out, deg_out,
        src_v, dst_v, rows_v, ones_v, xs_sh, deg_sh,
        isem, gsemA, gsemB, ssemA, ssemB, asem):
    c = lax.axis_index("c")
    s = lax.axis_index("s")

    r0 = s * ROWS_PER_SUB
    pltpu.sync_copy(z64_hbm, xs_sh.at[pl.ds(r0, ROWS_PER_SUB)])
    pltpu.sync_copy(z16_hbm, deg_sh.at[pl.ds(r0, ROWS_PER_SUB)])
    pltpu.sync_copy(ones_hbm, ones_v)
    pltpu.sync_copy(src_hbm.at[s], src_v)

    def mk_idx(j, carry):
      for kk in range(SUB // 16):
        sl = pl.ds(kk * 16, 16)
        src_v[j, sl] = src_v[j, sl] * 2 + c
      return carry

    lax.fori_loop(0, NSUB, mk_idx, 0)
    plsc.subcore_barrier()

    def body(i, carry):
      dld = pltpu.async_copy(dst_hbm.at[s, i], dst_v, isem)
      gA = [pltpu.async_copy(nf2_hbm.at[src_v.at[i * U + u]],
                             rows_v.at[u], gsemA)
            for u in range(UH)]
      gB = [pltpu.async_copy(nf2_hbm.at[src_v.at[i * U + u]],
                             rows_v.at[u], gsemB)
            for u in range(UH, U)]
      dld.wait()
      for d in gA:
        d.wait()

      sA = [pltpu.async_copy(rows_v.at[u], xs_sh.at[dst_v.at[u]],
                             ssemA, add=True)
            for u in range(UH)]

      @pl.when(c == 0)
      def _():
        dds = [pltpu.async_copy(ones_v, deg_sh.at[dst_v.at[u]],
                                asem, add=True)
               for u in range(UH)]
        for d in dds:
          d.wait()

      @pl.when(c == 1)
      def _():
        dds = [pltpu.async_copy(ones_v, deg_sh.at[dst_v.at[u]],
                                asem, add=True)
               for u in range(UH, U)]
        for d in dds:
          d.wait()

      for d in gB:
        d.wait()
      sB = [pltpu.async_copy(rows_v.at[u], xs_sh.at[dst_v.at[u]],
                             ssemB, add=True)
            for u in range(UH, U)]
      for d in sA:
        d.wait()
      for d in sB:
        d.wait()
      return carry

    lax.fori_loop(0, NBODY, body, 0)
    plsc.subcore_barrier()
    pltpu.sync_copy(xs_sh.at[pl.ds(r0, ROWS_PER_SUB)],
                    xs_out.at[c, pl.ds(r0, ROWS_PER_SUB)])
    pltpu.sync_copy(deg_sh.at[pl.ds(r0, ROWS_PER_SUB)],
                    deg_out.at[c, pl.ds(r0, ROWS_PER_SUB)])

  return k(src4d, dst4d, nf2, z64, z16, ones16)


def _sc_edge_path(dst4d, ef, z16):
  @functools.partial(
      pl.kernel,
      out_type=jax.ShapeDtypeStruct((NC, NPAD, DE), jnp.float32),
      mesh=_MESH,
      compiler_params=_SC_PARAMS,
      scratch_types=[
          pltpu.VMEM((U, SUB), jnp.int32),
          pltpu.VMEM((U, SUB, DE), jnp.float32),
          pltpu.VMEM_SHARED((NPAD, DE), jnp.float32),
          pltpu.SemaphoreType.DMA,
          pltpu.SemaphoreType.DMA,
          pltpu.SemaphoreType.DMA,
      ],
  )
  def k(dst_hbm, ef_hbm, z16_hbm, es_out,
        dst_v, ef_v, es_sh, isem, esem, asem):
    c = lax.axis_index("c")
    s = lax.axis_index("s")

    r0 = s * ROWS_PER_SUB
    pltpu.sync_copy(z16_hbm, es_sh.at[pl.ds(r0, ROWS_PER_SUB)])
    plsc.subcore_barrier()

    def body(i, carry):
      ib = 2 * i + c

      @pl.when(ib < NBODY)
      def _():
        ld = pltpu.async_copy(dst_hbm.at[s, ib], dst_v, isem)
        eds = [pltpu.async_copy(
                   ef_hbm.at[pl.ds((s * NSUB + ib * U + u) * SUB, SUB)],
                   ef_v.at[u], esem)
               for u in range(U)]
        ld.wait()
        for d in eds:
          d.wait()
        ads = [pltpu.async_copy(ef_v.at[u], es_sh.at[dst_v.at[u]],
                                asem, add=True)
               for u in range(U)]
        for d in ads:
          d.wait()

      return carry

    lax.fori_loop(0, (NBODY + 1) // 2, body, 0)
    plsc.subcore_barrier()
    pltpu.sync_copy(es_sh.at[pl.ds(r0, ROWS_PER_SUB)],
                    es_out.at[c, pl.ds(r0, ROWS_PER_SUB)])

  return k(dst4d, ef, z16)


_R = 2000


def _tc_xs_body(xs_ref, Wn_ref, Wm_ref, tmp_ref):
  f32 = jnp.float32
  hi = lax.Precision.HIGHEST
  Wq = lax.dot_general(Wm_ref[...], Wn_ref[...], (((1,), (0,)), ((), ())),
                       precision=hi, preferred_element_type=f32)
  num = lax.dot_general(xs_ref[0], Wq[:, :DH], (((1,), (1,)), ((), ())),
                        precision=hi, preferred_element_type=f32)
  num += lax.dot_general(xs_ref[1], Wq[:, DH:], (((1,), (1,)), ((), ())),
                         precision=hi, preferred_element_type=f32)
  tmp_ref[...] = num


def _tc_fin_body(tmp_ref, es_ref, deg_ref, We_ref, Wm_ref,
                 bn_ref, be_ref, bm_ref, out_ref):
  f32 = jnp.float32
  hi = lax.Precision.HIGHEST
  Wm = Wm_ref[...]
  Wr = lax.dot_general(Wm, We_ref[...], (((1,), (0,)), ((), ())),
                       precision=hi, preferred_element_type=f32)
  bsum = (bn_ref[...] + be_ref[...])[None, :]
  cvec = lax.dot_general(bsum, Wm, (((1,), (1,)), ((), ())),
                         precision=hi, preferred_element_type=f32)
  cvec = cvec + bm_ref[...][None, :]

  es = es_ref[0] + es_ref[1]
  deg = jnp.max(deg_ref[0] + deg_ref[1], axis=1, keepdims=True)

  num = tmp_ref[...]
  num += lax.dot_general(es, Wr, (((1,), (1,)), ((), ())),
                         precision=hi, preferred_element_type=f32)
  num += deg * cvec
  out_ref[...] = num / jnp.maximum(deg, 1.0)


def kernel(node_features, edge_features, edge_index, Wn, bn, We, be, Wm, bm):
  src3d = edge_index[0].reshape(NS, NSUB, SUB)
  dst4d = edge_index[1].reshape(NS, NBODY, U, SUB)
  nf2 = node_features.reshape(2 * N, DH)
  z64 = jnp.zeros((ROWS_PER_SUB, DH), jnp.float32)
  z16 = jnp.zeros((ROWS_PER_SUB, DE), jnp.float32)
  ones16 = jnp.ones((SUB, DE), jnp.float32)

  xs_p, deg_p = _sc_node_path(src3d, dst4d, nf2, z64, z16, ones16)
  es_p = _sc_edge_path(dst4d, edge_features, z16)

  tmp = pl.pallas_call(
      _tc_xs_body,
      grid=(N // _R,),
      in_specs=[
          pl.BlockSpec((NC, _R, DH), lambda i: (0, i, 0)),
          pl.BlockSpec((D, D), lambda i: (0, 0)),
          pl.BlockSpec((D, D), lambda i: (0, 0)),
      ],
      out_specs=pl.BlockSpec((_R, D), lambda i: (i, 0)),
      out_shape=jax.ShapeDtypeStruct((N, D), jnp.float32),
  )(xs_p, Wn, Wm)

  out = pl.pallas_call(
      _tc_fin_body,
      grid=(N // _R,),
      in_specs=[
          pl.BlockSpec((_R, D), lambda i: (i, 0)),
          pl.BlockSpec((NC, _R, DE), lambda i: (0, i, 0)),
          pl.BlockSpec((NC, _R, DE), lambda i: (0, i, 0)),
          pl.BlockSpec((D, DE), lambda i: (0, 0)),
          pl.BlockSpec((D, D), lambda i: (0, 0)),
          pl.BlockSpec((D,), lambda i: (0,)),
          pl.BlockSpec((D,), lambda i: (0,)),
          pl.BlockSpec((D,), lambda i: (0,)),
      ],
      out_specs=pl.BlockSpec((_R, D), lambda i: (i, 0)),
      out_shape=jax.ShapeDtypeStruct((N, D), jnp.float32),
  )(tmp, es_p, deg_p, We, Wm, bn, be, bm)
  return out
